# baseline (speedup 1.0000x reference)
import functools

import jax
import jax.numpy as jnp
from jax import lax
from jax.experimental import pallas as pl
from jax.experimental.pallas import tpu as pltpu
from jax.experimental.pallas import tpu_sc as plsc

N_NODES = 10000
N_EDGES = 320000
DIM = 128
EDGE_DIM = 16

NC = 2
NS = 16
NW = NC * NS
CHUNK = 48
SPLIT = 153600
NCH_A = 100
NCH_B = 108
ETAIL = 16
RPT = 624
TAIL = N_NODES - NS * RPT


def _xw1_body(x_ref, w_ref, b_ref, o_ref):
    o_ref[...] = (
        jnp.dot(x_ref[...], w_ref[...], preferred_element_type=jnp.float32)
        + b_ref[...]
    )


def _efw_body(et_ref, w_ref, o_ref):
    o_ref[...] = lax.dot_general(
        et_ref[...], w_ref[...],
        dimension_numbers=(((0,), (0,)), ((), ())),
        preferred_element_type=jnp.float32)


def _final_body(x_ref, a2a_ref, a2b_ref, c_ref, w2_ref, b2_ref, w3x_ref,
                w3a_ref, b3_ref, g_ref, be_ref, o_ref):
    agg_h = a2a_ref[0] + a2a_ref[1] + a2b_ref[0] + a2b_ref[1]
    cnt = jnp.sum(c_ref[...], axis=1, keepdims=True)
    agg = (jnp.dot(agg_h, w2_ref[...], preferred_element_type=jnp.float32)
           + cnt * b2_ref[...]) / (cnt + 1e-8)
    u = jnp.dot(x_ref[...], w3x_ref[...], preferred_element_type=jnp.float32)
    u = u + jnp.dot(agg, w3a_ref[...], preferred_element_type=jnp.float32)
    u = u + b3_ref[...]
    y = jnp.maximum(u, 0.0) + x_ref[...]
    m = jnp.mean(y, axis=1, keepdims=True)
    v = jnp.mean(jnp.square(y - m), axis=1, keepdims=True)
    y = (y - m) * lax.rsqrt(v + 1e-5)
    o_ref[...] = y * g_ref[...] + be_ref[...]


def _make_edge_body(nchunks, with_tail):
  epw = nchunks * CHUNK
  def _edge_body(xw1, efw, src, dst, zacc,
               agg_out, cnt_out,
               idx_s, idx_d, idxT_s, idxT_d, rows_v, ef_v, cnt_v, acc_s,
               sem_g0, sem_g1, sem_e0, sem_e1,
               sem_is0, sem_is1, sem_id0, sem_id1):
    c = lax.axis_index("c")
    s = lax.axis_index("s")
    wid = s * NC + c
    sem_g = (sem_g0, sem_g1)
    sem_e = (sem_e0, sem_e1)
    sem_is = (sem_is0, sem_is1)
    sem_id = (sem_id0, sem_id1)

    pltpu.sync_copy(zacc.at[pl.ds(s * RPT, RPT)], acc_s.at[pl.ds(s * RPT, RPT)])

    @pl.when(s == NS - 1)
    def _zero_tail():
        pltpu.sync_copy(zacc.at[pl.ds(NS * RPT, TAIL)],
                        acc_s.at[pl.ds(NS * RPT, TAIL)])

    zero16 = jnp.zeros((16,), jnp.float32)

    def zero_body(i, _):
        cnt_v[pl.ds(i * 16, 16)] = zero16
        return ()

    lax.fori_loop(0, N_NODES // 16, zero_body, (), unroll=8)
    plsc.subcore_barrier()

    one16 = jnp.full((16,), 1.0, jnp.float32)

    ebase = wid * epw

    def start_idx_s(t, b):
        pltpu.async_copy(src.at[pl.ds(ebase + t * CHUNK, CHUNK)],
                         idx_s.at[b], sem_is[b])

    def start_idx_d(t, b):
        pltpu.async_copy(dst.at[pl.ds(ebase + t * CHUNK, CHUNK)],
                         idx_d.at[b], sem_id[b])

    def wait_idx(t, b):
        pltpu.make_async_copy(src.at[pl.ds(ebase + t * CHUNK, CHUNK)],
                              idx_s.at[b], sem_is[b]).wait()
        pltpu.make_async_copy(dst.at[pl.ds(ebase + t * CHUNK, CHUNK)],
                              idx_d.at[b], sem_id[b]).wait()

    def start(t, b):
        pltpu.async_copy(efw.at[pl.ds(ebase + t * CHUNK, CHUNK)],
                         ef_v.at[b], sem_e[b])
        pltpu.async_copy(xw1.at[idx_s.at[b]], rows_v.at[b], sem_g[b])

    def finish(t, b):
        pltpu.make_async_copy(efw.at[pl.ds(ebase + t * CHUNK, CHUNK)],
                              ef_v.at[b], sem_e[b]).wait()
        pltpu.make_async_copy(xw1.at[idx_s.at[b]], rows_v.at[b],
                              sem_g[b]).wait()

    def compute(t, b):
        @plsc.parallel_loop(0, CHUNK, step=1, unroll=4)
        def _relu(i):
            for j in range(DIM // 16):
                sl = pl.ds(j * 16, 16)
                v = rows_v[b, i, sl] + ef_v[b, i, sl]
                rows_v[b, i, sl] = jnp.maximum(v, 0.0)

        pltpu.sync_copy(rows_v.at[b], acc_s.at[idx_d.at[b]], add=True)

        for k in range(CHUNK // 16):
            idx = idx_d[b, pl.ds(k * 16, 16)]
            plsc.addupdate_scatter(cnt_v, [idx], one16)

    start_idx_s(0, 0)
    start_idx_d(0, 0)
    start_idx_s(1, 1)
    start_idx_d(1, 1)
    wait_idx(0, 0)
    start(0, 0)

    def pair_body(u, _):
        t0 = 2 * u
        last = nchunks // 2 - 1

        finish(t0, 0)

        @pl.when(u < last)
        def _():
            start_idx_s(t0 + 2, 0)

        wait_idx(t0 + 1, 1)
        start(t0 + 1, 1)
        compute(t0, 0)

        @pl.when(u < last)
        def _():
            start_idx_d(t0 + 2, 0)

        finish(t0 + 1, 1)

        @pl.when(u < last)
        def _():
            start_idx_s(t0 + 3, 1)
            wait_idx(t0 + 2, 0)
            start(t0 + 2, 0)

        compute(t0 + 1, 1)

        @pl.when(u < last)
        def _():
            start_idx_d(t0 + 3, 1)

        return ()

    lax.fori_loop(0, nchunks // 2, pair_body, ())

    if with_tail:
        tbase = NW * epw + wid * ETAIL
        pltpu.sync_copy(src.at[pl.ds(tbase, ETAIL)], idxT_s)
        pltpu.sync_copy(dst.at[pl.ds(tbase, ETAIL)], idxT_d)
        pltpu.sync_copy(efw.at[pl.ds(tbase, ETAIL)],
                        ef_v.at[0, pl.ds(0, ETAIL)])
        pltpu.async_copy(xw1.at[idxT_s], rows_v.at[0, pl.ds(0, ETAIL)],
                         sem_g0).wait()

        def tail_body(i, _):
            for j in range(DIM // 16):
                sl = pl.ds(j * 16, 16)
                v = rows_v[0, i, sl] + ef_v[0, i, sl]
                rows_v[0, i, sl] = jnp.maximum(v, 0.0)
            return ()

        lax.fori_loop(0, ETAIL, tail_body, ())
        pltpu.sync_copy(rows_v.at[0, pl.ds(0, ETAIL)], acc_s.at[idxT_d],
                        add=True)
        plsc.addupdate_scatter(cnt_v, [idxT_d[...]], one16)

    plsc.subcore_barrier()

    pltpu.sync_copy(acc_s.at[pl.ds(s * RPT, RPT)],
                    agg_out.at[c, pl.ds(s * RPT, RPT)])

    @pl.when(s == NS - 1)
    def _write_tail():
        pltpu.sync_copy(acc_s.at[pl.ds(NS * RPT, TAIL)],
                        agg_out.at[c, pl.ds(NS * RPT, TAIL)])

    pltpu.sync_copy(cnt_v, cnt_out.at[pl.ds(wid * N_NODES, N_NODES)])

  return _edge_body


def kernel(x, edge_index, edge_features, W1, b1, W2, b2, W3, b3, gamma, beta):
    x = x.astype(jnp.float32)
    src = edge_index[0].astype(jnp.int32)
    dst = edge_index[1].astype(jnp.int32)

    w1xT = W1[:, :DIM].T
    w1eT = W1[:, DIM:].T
    w2T = W2.T
    w3xT = W3[:, :DIM].T
    w3aT = W3[:, DIM:].T
    b1r = b1.reshape(1, DIM)
    b2r = b2.reshape(1, DIM)
    b3r = b3.reshape(1, DIM)
    gr = gamma.reshape(1, DIM)
    br = beta.reshape(1, DIM)

    xw1 = pl.pallas_call(
        _xw1_body,
        grid=(5,),
        in_specs=[
            pl.BlockSpec((2000, DIM), lambda i: (i, 0)),
            pl.BlockSpec((DIM, DIM), lambda i: (0, 0)),
            pl.BlockSpec((1, DIM), lambda i: (0, 0)),
        ],
        out_specs=pl.BlockSpec((2000, DIM), lambda i: (i, 0)),
        out_shape=jax.ShapeDtypeStruct((N_NODES, DIM), jnp.float32),
    )(x, w1xT, b1r)

    eft = edge_features.T
    efwA = pl.pallas_call(
        _efw_body,
        grid=(12,),
        in_specs=[
            pl.BlockSpec((EDGE_DIM, 12800), lambda i: (0, i)),
            pl.BlockSpec((EDGE_DIM, DIM), lambda i: (0, 0)),
        ],
        out_specs=pl.BlockSpec((12800, DIM), lambda i: (i, 0)),
        out_shape=jax.ShapeDtypeStruct((SPLIT, DIM), jnp.float32),
    )(eft, w1eT)
    efwB = pl.pallas_call(
        _efw_body,
        grid=(26,),
        in_specs=[
            pl.BlockSpec((EDGE_DIM, 6400), lambda i: (0, i + 24)),
            pl.BlockSpec((EDGE_DIM, DIM), lambda i: (0, 0)),
        ],
        out_specs=pl.BlockSpec((6400, DIM), lambda i: (i, 0)),
        out_shape=jax.ShapeDtypeStruct((N_EDGES - SPLIT, DIM), jnp.float32),
    )(eft, w1eT)

    zacc = jnp.zeros((N_NODES, DIM), jnp.float32)

    def sc_call(nchunks, with_tail, xw1_, efw_, src_, dst_):
        return pl.kernel(
            _make_edge_body(nchunks, with_tail),
            out_type=(
                jax.ShapeDtypeStruct((NC, N_NODES, DIM), jnp.float32),
                jax.ShapeDtypeStruct((NW * N_NODES,), jnp.float32),
            ),
            mesh=plsc.VectorSubcoreMesh(core_axis_name="c",
                                        subcore_axis_name="s"),
            compiler_params=pltpu.CompilerParams(needs_layout_passes=False),
            scratch_types=[
                pltpu.VMEM((2, CHUNK), jnp.int32),
                pltpu.VMEM((2, CHUNK), jnp.int32),
                pltpu.VMEM((ETAIL,), jnp.int32),
                pltpu.VMEM((ETAIL,), jnp.int32),
                pltpu.VMEM((2, CHUNK, DIM), jnp.float32),
                pltpu.VMEM((2, CHUNK, DIM), jnp.float32),
                pltpu.VMEM((N_NODES,), jnp.float32),
                pltpu.VMEM_SHARED((N_NODES, DIM), jnp.float32),
            ] + [pltpu.SemaphoreType.DMA] * 8,
        )(xw1_, efw_, src_, dst_, zacc)

    agg2A, cntA = sc_call(NCH_A, False, xw1, efwA,
                          src[:SPLIT], dst[:SPLIT])
    agg2B, cntB = sc_call(NCH_B, True, xw1, efwB,
                          src[SPLIT:], dst[SPLIT:])
    cnt32 = (cntA.reshape(NW, N_NODES) + cntB.reshape(NW, N_NODES))

    y = pl.pallas_call(
        _final_body,
        grid=(5,),
        in_specs=[
            pl.BlockSpec((2000, DIM), lambda i: (i, 0)),
            pl.BlockSpec((NC, 2000, DIM), lambda i: (0, i, 0)),
            pl.BlockSpec((NC, 2000, DIM), lambda i: (0, i, 0)),
            pl.BlockSpec((2000, NW), lambda i: (i, 0)),
            pl.BlockSpec((DIM, DIM), lambda i: (0, 0)),
            pl.BlockSpec((1, DIM), lambda i: (0, 0)),
            pl.BlockSpec((DIM, DIM), lambda i: (0, 0)),
            pl.BlockSpec((DIM, DIM), lambda i: (0, 0)),
            pl.BlockSpec((1, DIM), lambda i: (0, 0)),
            pl.BlockSpec((1, DIM), lambda i: (0, 0)),
            pl.BlockSpec((1, DIM), lambda i: (0, 0)),
        ],
        out_specs=pl.BlockSpec((2000, DIM), lambda i: (i, 0)),
        out_shape=jax.ShapeDtypeStruct((N_NODES, DIM), jnp.float32),
    )(x, agg2A, agg2B, cnt32.T, w2T, b2r, w3xT, w3aT, b3r, gr, br)

    return y


# speedup vs baseline: 1.0394x; 1.0394x over previous
import functools

import jax
import jax.numpy as jnp
from jax import lax
from jax.experimental import pallas as pl
from jax.experimental.pallas import tpu as pltpu
from jax.experimental.pallas import tpu_sc as plsc

N_NODES = 10000
N_EDGES = 320000
DIM = 128
EDGE_DIM = 16

NC = 2
NS = 16
NW = NC * NS
CHUNK = 48
NCHUNKS = 208
EPW = NCHUNKS * CHUNK
ETAIL = (N_EDGES - NW * EPW) // NW
TAIL_BASE = NW * EPW
RPT = 624
TAIL = N_NODES - NS * RPT


def _xw1_body(x_ref, w_ref, b_ref, o_ref):
    o_ref[...] = (
        jnp.dot(x_ref[...], w_ref[...], preferred_element_type=jnp.float32)
        + b_ref[...]
    )


def _efw_body(et_ref, w_ref, o_ref):
    o_ref[...] = lax.dot_general(
        et_ref[...], w_ref[...],
        dimension_numbers=(((0,), (0,)), ((), ())),
        preferred_element_type=jnp.float32)


def _final_body(x_ref, a2_ref, c_ref, w2_ref, b2_ref, w3x_ref, w3a_ref,
                b3_ref, g_ref, be_ref, o_ref):
    agg_h = a2_ref[0] + a2_ref[1]
    cnt = jnp.sum(c_ref[...], axis=1, keepdims=True)
    agg = (jnp.dot(agg_h, w2_ref[...], preferred_element_type=jnp.float32)
           + cnt * b2_ref[...]) / (cnt + 1e-8)
    u = jnp.dot(x_ref[...], w3x_ref[...], preferred_element_type=jnp.float32)
    u = u + jnp.dot(agg, w3a_ref[...], preferred_element_type=jnp.float32)
    u = u + b3_ref[...]
    y = jnp.maximum(u, 0.0) + x_ref[...]
    m = jnp.mean(y, axis=1, keepdims=True)
    v = jnp.mean(jnp.square(y - m), axis=1, keepdims=True)
    y = (y - m) * lax.rsqrt(v + 1e-5)
    o_ref[...] = y * g_ref[...] + be_ref[...]


def _edge_body(xw1, efw, src, dst, zacc,
               agg_out, cnt_out,
               idx_s, idx_d, idxT_s, idxT_d, rows_v, ef_v, cnt_v, acc_s,
               sem_g0, sem_g1, sem_e0, sem_e1,
               sem_is0, sem_is1, sem_id0, sem_id1):
    c = lax.axis_index("c")
    s = lax.axis_index("s")
    wid = s * NC + c
    sem_g = (sem_g0, sem_g1)
    sem_e = (sem_e0, sem_e1)
    sem_is = (sem_is0, sem_is1)
    sem_id = (sem_id0, sem_id1)

    pltpu.sync_copy(zacc.at[pl.ds(s * RPT, RPT)], acc_s.at[pl.ds(s * RPT, RPT)])

    @pl.when(s == NS - 1)
    def _zero_tail():
        pltpu.sync_copy(zacc.at[pl.ds(NS * RPT, TAIL)],
                        acc_s.at[pl.ds(NS * RPT, TAIL)])

    zero16 = jnp.zeros((16,), jnp.float32)

    def zero_body(i, _):
        cnt_v[pl.ds(i * 16, 16)] = zero16
        return ()

    lax.fori_loop(0, N_NODES // 16, zero_body, (), unroll=8)
    plsc.subcore_barrier()

    one16 = jnp.full((16,), 1.0, jnp.float32)

    ebase = wid * EPW

    def start_idx_s(t, b):
        pltpu.async_copy(src.at[pl.ds(ebase + t * CHUNK, CHUNK)],
                         idx_s.at[b], sem_is[b])

    def start_idx_d(t, b):
        pltpu.async_copy(dst.at[pl.ds(ebase + t * CHUNK, CHUNK)],
                         idx_d.at[b], sem_id[b])

    def wait_idx(t, b):
        pltpu.make_async_copy(src.at[pl.ds(ebase + t * CHUNK, CHUNK)],
                              idx_s.at[b], sem_is[b]).wait()
        pltpu.make_async_copy(dst.at[pl.ds(ebase + t * CHUNK, CHUNK)],
                              idx_d.at[b], sem_id[b]).wait()

    def start(t, b):
        pltpu.async_copy(efw.at[pl.ds(ebase + t * CHUNK, CHUNK)],
                         ef_v.at[b], sem_e[b])
        pltpu.async_copy(xw1.at[idx_s.at[b]], rows_v.at[b], sem_g[b])

    def finish(t, b):
        pltpu.make_async_copy(efw.at[pl.ds(ebase + t * CHUNK, CHUNK)],
                              ef_v.at[b], sem_e[b]).wait()
        pltpu.make_async_copy(xw1.at[idx_s.at[b]], rows_v.at[b],
                              sem_g[b]).wait()

    def compute(t, b):
        @plsc.parallel_loop(0, CHUNK, step=1, unroll=8)
        def _relu(i):
            for j in range(DIM // 16):
                sl = pl.ds(j * 16, 16)
                v = rows_v[b, i, sl] + ef_v[b, i, sl]
                rows_v[b, i, sl] = jnp.maximum(v, 0.0)

        pltpu.sync_copy(rows_v.at[b], acc_s.at[idx_d.at[b]], add=True)

        for k in range(CHUNK // 16):
            idx = idx_d[b, pl.ds(k * 16, 16)]
            plsc.addupdate_scatter(cnt_v, [idx], one16)

    start_idx_s(0, 0)
    start_idx_d(0, 0)
    start_idx_s(1, 1)
    start_idx_d(1, 1)
    wait_idx(0, 0)
    start(0, 0)

    def pair_body(u, _):
        t0 = 2 * u
        last = NCHUNKS // 2 - 1

        finish(t0, 0)

        @pl.when(u < last)
        def _():
            start_idx_s(t0 + 2, 0)

        wait_idx(t0 + 1, 1)
        start(t0 + 1, 1)
        compute(t0, 0)

        @pl.when(u < last)
        def _():
            start_idx_d(t0 + 2, 0)

        finish(t0 + 1, 1)

        @pl.when(u < last)
        def _():
            start_idx_s(t0 + 3, 1)
            wait_idx(t0 + 2, 0)
            start(t0 + 2, 0)

        compute(t0 + 1, 1)

        @pl.when(u < last)
        def _():
            start_idx_d(t0 + 3, 1)

        return ()

    lax.fori_loop(0, NCHUNKS // 2, pair_body, ())

    tbase = TAIL_BASE + wid * ETAIL
    pltpu.sync_copy(src.at[pl.ds(tbase, ETAIL)], idxT_s)
    pltpu.sync_copy(dst.at[pl.ds(tbase, ETAIL)], idxT_d)
    pltpu.sync_copy(efw.at[pl.ds(tbase, ETAIL)],
                    ef_v.at[0, pl.ds(0, ETAIL)])
    pltpu.async_copy(xw1.at[idxT_s], rows_v.at[0, pl.ds(0, ETAIL)],
                     sem_g0).wait()

    def tail_body(i, _):
        for j in range(DIM // 16):
            sl = pl.ds(j * 16, 16)
            v = rows_v[0, i, sl] + ef_v[0, i, sl]
            rows_v[0, i, sl] = jnp.maximum(v, 0.0)
        return ()

    lax.fori_loop(0, ETAIL, tail_body, ())
    pltpu.sync_copy(rows_v.at[0, pl.ds(0, ETAIL)], acc_s.at[idxT_d], add=True)
    plsc.addupdate_scatter(cnt_v, [idxT_d[...]], one16)

    plsc.subcore_barrier()

    pltpu.sync_copy(acc_s.at[pl.ds(s * RPT, RPT)],
                    agg_out.at[c, pl.ds(s * RPT, RPT)])

    @pl.when(s == NS - 1)
    def _write_tail():
        pltpu.sync_copy(acc_s.at[pl.ds(NS * RPT, TAIL)],
                        agg_out.at[c, pl.ds(NS * RPT, TAIL)])

    pltpu.sync_copy(cnt_v, cnt_out.at[pl.ds(wid * N_NODES, N_NODES)])


def kernel(x, edge_index, edge_features, W1, b1, W2, b2, W3, b3, gamma, beta):
    x = x.astype(jnp.float32)
    src = edge_index[0].astype(jnp.int32)
    dst = edge_index[1].astype(jnp.int32)

    w1xT = W1[:, :DIM].T
    w1eT = W1[:, DIM:].T
    w2T = W2.T
    w3xT = W3[:, :DIM].T
    w3aT = W3[:, DIM:].T
    b1r = b1.reshape(1, DIM)
    b2r = b2.reshape(1, DIM)
    b3r = b3.reshape(1, DIM)
    gr = gamma.reshape(1, DIM)
    br = beta.reshape(1, DIM)

    xw1 = pl.pallas_call(
        _xw1_body,
        grid=(5,),
        in_specs=[
            pl.BlockSpec((2000, DIM), lambda i: (i, 0)),
            pl.BlockSpec((DIM, DIM), lambda i: (0, 0)),
            pl.BlockSpec((1, DIM), lambda i: (0, 0)),
        ],
        out_specs=pl.BlockSpec((2000, DIM), lambda i: (i, 0)),
        out_shape=jax.ShapeDtypeStruct((N_NODES, DIM), jnp.float32),
    )(x, w1xT, b1r)

    efw = pl.pallas_call(
        _efw_body,
        grid=(20,),
        in_specs=[
            pl.BlockSpec((EDGE_DIM, 16000), lambda i: (0, i)),
            pl.BlockSpec((EDGE_DIM, DIM), lambda i: (0, 0)),
        ],
        out_specs=pl.BlockSpec((16000, DIM), lambda i: (i, 0)),
        out_shape=jax.ShapeDtypeStruct((N_EDGES, DIM), jnp.float32),
    )(edge_features.T, w1eT)

    zacc = jnp.zeros((N_NODES, DIM), jnp.float32)

    agg2, cnt_flat = pl.kernel(
        _edge_body,
        out_type=(
            jax.ShapeDtypeStruct((NC, N_NODES, DIM), jnp.float32),
            jax.ShapeDtypeStruct((NW * N_NODES,), jnp.float32),
        ),
        mesh=plsc.VectorSubcoreMesh(core_axis_name="c", subcore_axis_name="s"),
        compiler_params=pltpu.CompilerParams(needs_layout_passes=False),
        scratch_types=[
            pltpu.VMEM((2, CHUNK), jnp.int32),
            pltpu.VMEM((2, CHUNK), jnp.int32),
            pltpu.VMEM((ETAIL,), jnp.int32),
            pltpu.VMEM((ETAIL,), jnp.int32),
            pltpu.VMEM((2, CHUNK, DIM), jnp.float32),
            pltpu.VMEM((2, CHUNK, DIM), jnp.float32),
            pltpu.VMEM((N_NODES,), jnp.float32),
            pltpu.VMEM_SHARED((N_NODES, DIM), jnp.float32),
        ] + [pltpu.SemaphoreType.DMA] * 8,
    )(xw1, efw, src, dst, zacc)
    cnt32 = cnt_flat.reshape(NW, N_NODES)

    y = pl.pallas_call(
        _final_body,
        grid=(5,),
        in_specs=[
            pl.BlockSpec((2000, DIM), lambda i: (i, 0)),
            pl.BlockSpec((NC, 2000, DIM), lambda i: (0, i, 0)),
            pl.BlockSpec((2000, NW), lambda i: (i, 0)),
            pl.BlockSpec((DIM, DIM), lambda i: (0, 0)),
            pl.BlockSpec((1, DIM), lambda i: (0, 0)),
            pl.BlockSpec((DIM, DIM), lambda i: (0, 0)),
            pl.BlockSpec((DIM, DIM), lambda i: (0, 0)),
            pl.BlockSpec((1, DIM), lambda i: (0, 0)),
            pl.BlockSpec((1, DIM), lambda i: (0, 0)),
            pl.BlockSpec((1, DIM), lambda i: (0, 0)),
        ],
        out_specs=pl.BlockSpec((2000, DIM), lambda i: (i, 0)),
        out_shape=jax.ShapeDtypeStruct((N_NODES, DIM), jnp.float32),
    )(x, agg2, cnt32.T, w2T, b2r, w3xT, w3aT, b3r, gr, br)

    return y


# speedup vs baseline: 1.0403x; 1.0008x over previous
import jax
import jax.numpy as jnp
from jax import lax
from jax.experimental import pallas as pl
from jax.experimental.pallas import tpu as pltpu
from jax.experimental.pallas import tpu_sc as plsc

N_NODES = 10000
N_EDGES = 320000
DIM = 128
EDGE_DIM = 16

NC = 2
NS = 16
NW = NC * NS
CHUNK = 48
NCHUNKS = 208
EPW = NCHUNKS * CHUNK
ETAIL = (N_EDGES - NW * EPW) // NW
TAIL_BASE = NW * EPW
RPT = 624
TAIL = N_NODES - NS * RPT


def _xw1_body(x_ref, w_ref, b_ref, o_ref):
    o_ref[...] = (
        jnp.dot(x_ref[...], w_ref[...], preferred_element_type=jnp.float32)
        + b_ref[...]
    )


def _efw_body(et_ref, w_ref, o_ref):
    o_ref[...] = lax.dot_general(
        et_ref[...], w_ref[...],
        dimension_numbers=(((0,), (0,)), ((), ())),
        preferred_element_type=jnp.float32)


def _final_body(x_ref, a2_ref, c_ref, w2_ref, b2_ref, w3x_ref, w3a_ref,
                b3_ref, g_ref, be_ref, o_ref):
    agg_h = a2_ref[0] + a2_ref[1]
    cnt = jnp.sum(c_ref[...], axis=1, keepdims=True)
    agg = (jnp.dot(agg_h, w2_ref[...], preferred_element_type=jnp.float32)
           + cnt * b2_ref[...]) / (cnt + 1e-8)
    u = jnp.dot(x_ref[...], w3x_ref[...], preferred_element_type=jnp.float32)
    u = u + jnp.dot(agg, w3a_ref[...], preferred_element_type=jnp.float32)
    u = u + b3_ref[...]
    y = jnp.maximum(u, 0.0) + x_ref[...]
    m = jnp.mean(y, axis=1, keepdims=True)
    v = jnp.mean(jnp.square(y - m), axis=1, keepdims=True)
    y = (y - m) * lax.rsqrt(v + 1e-5)
    o_ref[...] = y * g_ref[...] + be_ref[...]


def _edge_body(xw1, efw, src, dst, zacc,
               agg_out, cnt_out,
               idx_s, idx_d, idxT_s, idxT_d, rows_v, ef_v, cnt_v, acc_s,
               sem_g0, sem_g1, sem_e0, sem_e1,
               sem_is0, sem_is1, sem_id0, sem_id1):
    c = lax.axis_index("c")
    s = lax.axis_index("s")
    wid = s * NC + c
    sem_g = (sem_g0, sem_g1)
    sem_e = (sem_e0, sem_e1)
    sem_is = (sem_is0, sem_is1)
    sem_id = (sem_id0, sem_id1)

    pltpu.sync_copy(zacc.at[pl.ds(s * RPT, RPT)], acc_s.at[pl.ds(s * RPT, RPT)])

    @pl.when(s == NS - 1)
    def _zero_tail():
        pltpu.sync_copy(zacc.at[pl.ds(NS * RPT, TAIL)],
                        acc_s.at[pl.ds(NS * RPT, TAIL)])

    zero16 = jnp.zeros((16,), jnp.float32)

    def zero_body(i, _):
        cnt_v[pl.ds(i * 16, 16)] = zero16
        return ()

    lax.fori_loop(0, N_NODES // 16, zero_body, (), unroll=8)
    plsc.subcore_barrier()

    one16 = jnp.full((16,), 1.0, jnp.float32)

    ebase = wid * EPW

    def start_idx_s(t, b):
        pltpu.async_copy(src.at[pl.ds(ebase + t * CHUNK, CHUNK)],
                         idx_s.at[b], sem_is[b])

    def start_idx_d(t, b):
        pltpu.async_copy(dst.at[pl.ds(ebase + t * CHUNK, CHUNK)],
                         idx_d.at[b], sem_id[b])

    def wait_idx(t, b):
        pltpu.make_async_copy(src.at[pl.ds(ebase + t * CHUNK, CHUNK)],
                              idx_s.at[b], sem_is[b]).wait()
        pltpu.make_async_copy(dst.at[pl.ds(ebase + t * CHUNK, CHUNK)],
                              idx_d.at[b], sem_id[b]).wait()

    def start(t, b):
        pltpu.async_copy(efw.at[pl.ds(ebase + t * CHUNK, CHUNK)],
                         ef_v.at[b], sem_e[b])
        pltpu.async_copy(xw1.at[idx_s.at[b]], rows_v.at[b], sem_g[b])

    def finish(t, b):
        pltpu.make_async_copy(efw.at[pl.ds(ebase + t * CHUNK, CHUNK)],
                              ef_v.at[b], sem_e[b]).wait()
        pltpu.make_async_copy(xw1.at[idx_s.at[b]], rows_v.at[b],
                              sem_g[b]).wait()

    def compute(t, b):
        @plsc.parallel_loop(0, CHUNK, step=1, unroll=8)
        def _relu(i):
            for j in range(DIM // 16):
                sl = pl.ds(j * 16, 16)
                v = rows_v[b, i, sl] + ef_v[b, i, sl]
                rows_v[b, i, sl] = jnp.maximum(v, 0.0)

        pltpu.sync_copy(rows_v.at[b], acc_s.at[idx_d.at[b]], add=True)

        for k in range(CHUNK // 16):
            idx = idx_d[b, pl.ds(k * 16, 16)]
            plsc.addupdate_scatter(cnt_v, [idx], one16)

    start_idx_s(0, 0)
    start_idx_d(0, 0)
    start_idx_s(1, 1)
    start_idx_d(1, 1)
    wait_idx(0, 0)
    start(0, 0)

    def pair_body(u, _):
        t0 = 2 * u
        last = NCHUNKS // 2 - 1

        finish(t0, 0)

        @pl.when(u < last)
        def _():
            start_idx_s(t0 + 2, 0)

        wait_idx(t0 + 1, 1)
        start(t0 + 1, 1)
        compute(t0, 0)

        @pl.when(u < last)
        def _():
            start_idx_d(t0 + 2, 0)

        finish(t0 + 1, 1)

        @pl.when(u < last)
        def _():
            start_idx_s(t0 + 3, 1)
            wait_idx(t0 + 2, 0)
            start(t0 + 2, 0)

        compute(t0 + 1, 1)

        @pl.when(u < last)
        def _():
            start_idx_d(t0 + 3, 1)

        return ()

    lax.fori_loop(0, NCHUNKS // 2, pair_body, ())

    tbase = TAIL_BASE + wid * ETAIL
    pltpu.sync_copy(src.at[pl.ds(tbase, ETAIL)], idxT_s)
    pltpu.sync_copy(dst.at[pl.ds(tbase, ETAIL)], idxT_d)
    pltpu.sync_copy(efw.at[pl.ds(tbase, ETAIL)],
                    ef_v.at[0, pl.ds(0, ETAIL)])
    pltpu.async_copy(xw1.at[idxT_s], rows_v.at[0, pl.ds(0, ETAIL)],
                     sem_g0).wait()

    def tail_body(i, _):
        for j in range(DIM // 16):
            sl = pl.ds(j * 16, 16)
            v = rows_v[0, i, sl] + ef_v[0, i, sl]
            rows_v[0, i, sl] = jnp.maximum(v, 0.0)
        return ()

    lax.fori_loop(0, ETAIL, tail_body, ())
    pltpu.sync_copy(rows_v.at[0, pl.ds(0, ETAIL)], acc_s.at[idxT_d], add=True)
    plsc.addupdate_scatter(cnt_v, [idxT_d[...]], one16)

    plsc.subcore_barrier()

    pltpu.sync_copy(acc_s.at[pl.ds(s * RPT, RPT)],
                    agg_out.at[c, pl.ds(s * RPT, RPT)])

    @pl.when(s == NS - 1)
    def _write_tail():
        pltpu.sync_copy(acc_s.at[pl.ds(NS * RPT, TAIL)],
                        agg_out.at[c, pl.ds(NS * RPT, TAIL)])

    pltpu.sync_copy(cnt_v, cnt_out.at[pl.ds(wid * N_NODES, N_NODES)])


def kernel(x, edge_index, edge_features, W1, b1, W2, b2, W3, b3, gamma, beta):
    x = x.astype(jnp.float32)
    src = edge_index[0].astype(jnp.int32)
    dst = edge_index[1].astype(jnp.int32)

    w1xT = W1[:, :DIM].T
    w1eT = W1[:, DIM:].T
    w2T = W2.T
    w3xT = W3[:, :DIM].T
    w3aT = W3[:, DIM:].T
    b1r = b1.reshape(1, DIM)
    b2r = b2.reshape(1, DIM)
    b3r = b3.reshape(1, DIM)
    gr = gamma.reshape(1, DIM)
    br = beta.reshape(1, DIM)

    xw1 = pl.pallas_call(
        _xw1_body,
        grid=(5,),
        in_specs=[
            pl.BlockSpec((2000, DIM), lambda i: (i, 0)),
            pl.BlockSpec((DIM, DIM), lambda i: (0, 0)),
            pl.BlockSpec((1, DIM), lambda i: (0, 0)),
        ],
        out_specs=pl.BlockSpec((2000, DIM), lambda i: (i, 0)),
        out_shape=jax.ShapeDtypeStruct((N_NODES, DIM), jnp.float32),
    )(x, w1xT, b1r)

    efw = pl.pallas_call(
        _efw_body,
        grid=(20,),
        in_specs=[
            pl.BlockSpec((EDGE_DIM, 16000), lambda i: (0, i)),
            pl.BlockSpec((EDGE_DIM, DIM), lambda i: (0, 0)),
        ],
        out_specs=pl.BlockSpec((16000, DIM), lambda i: (i, 0)),
        out_shape=jax.ShapeDtypeStruct((N_EDGES, DIM), jnp.float32),
    )(edge_features.T, w1eT)

    zacc = jnp.zeros((N_NODES, DIM), jnp.float32)

    agg2, cnt_flat = pl.kernel(
        _edge_body,
        out_type=(
            jax.ShapeDtypeStruct((NC, N_NODES, DIM), jnp.float32),
            jax.ShapeDtypeStruct((NW * N_NODES,), jnp.float32),
        ),
        mesh=plsc.VectorSubcoreMesh(core_axis_name="c", subcore_axis_name="s"),
        compiler_params=pltpu.CompilerParams(needs_layout_passes=False),
        scratch_types=[
            pltpu.VMEM((2, CHUNK), jnp.int32),
            pltpu.VMEM((2, CHUNK), jnp.int32),
            pltpu.VMEM((ETAIL,), jnp.int32),
            pltpu.VMEM((ETAIL,), jnp.int32),
            pltpu.VMEM((2, CHUNK, DIM), jnp.float32),
            pltpu.VMEM((2, CHUNK, DIM), jnp.float32),
            pltpu.VMEM((N_NODES,), jnp.float32),
            pltpu.VMEM_SHARED((N_NODES, DIM), jnp.float32),
        ] + [pltpu.SemaphoreType.DMA] * 8,
    )(xw1, efw, src, dst, zacc)
    cnt32 = cnt_flat.reshape(NW, N_NODES)

    y = pl.pallas_call(
        _final_body,
        grid=(5,),
        in_specs=[
            pl.BlockSpec((2000, DIM), lambda i: (i, 0)),
            pl.BlockSpec((NC, 2000, DIM), lambda i: (0, i, 0)),
            pl.BlockSpec((2000, NW), lambda i: (i, 0)),
            pl.BlockSpec((DIM, DIM), lambda i: (0, 0)),
            pl.BlockSpec((1, DIM), lambda i: (0, 0)),
            pl.BlockSpec((DIM, DIM), lambda i: (0, 0)),
            pl.BlockSpec((DIM, DIM), lambda i: (0, 0)),
            pl.BlockSpec((1, DIM), lambda i: (0, 0)),
            pl.BlockSpec((1, DIM), lambda i: (0, 0)),
            pl.BlockSpec((1, DIM), lambda i: (0, 0)),
        ],
        out_specs=pl.BlockSpec((2000, DIM), lambda i: (i, 0)),
        out_shape=jax.ShapeDtypeStruct((N_NODES, DIM), jnp.float32),
    )(x, agg2, cnt32.T, w2T, b2r, w3xT, w3aT, b3r, gr, br)

    return y
